# Initial kernel scaffold; baseline (speedup 1.0000x reference)
import functools

import jax
import jax.numpy as jnp
from jax import lax
from jax.experimental import pallas as pl
from jax.experimental.pallas import tpu as pltpu
from jax.experimental.pallas import tpu_sc as plsc

NUM_TOKENS = 16384 * 50
EMB_DIM = 32

NC = 2
NS = 16
NW = NC * NS
PER_W = NUM_TOKENS // NW
CHUNK = 128
N_CHUNKS = PER_W // CHUNK
K = 20
N_GROUPS = N_CHUNKS // K


def _body(idx_hbm, w_hbm, out_hbm, idx_v, bufs, sem_g, sem_o):
    wid = lax.axis_index("s") * NC + lax.axis_index("c")
    pltpu.sync_copy(idx_hbm.at[wid], idx_v)

    def group(m, carry):
        base = m * K
        gathers = [
            pltpu.async_copy(w_hbm.at[idx_v.at[base + b]], bufs.at[b], sem_g)
            for b in range(K)
        ]
        for cp in gathers:
            cp.wait()
        outs = [
            pltpu.async_copy(bufs.at[b], out_hbm.at[wid, base + b], sem_o)
            for b in range(K)
        ]
        for cp in outs:
            cp.wait()
        return carry

    lax.fori_loop(0, N_GROUPS, group, 0)


@jax.jit
def _gather(idx, weight):
    mesh = plsc.VectorSubcoreMesh(core_axis_name="c", subcore_axis_name="s")
    fn = pl.kernel(
        _body,
        out_type=jax.ShapeDtypeStruct((NW, N_CHUNKS, CHUNK, EMB_DIM), jnp.float32),
        mesh=mesh,
        scratch_types=[
            pltpu.VMEM((N_CHUNKS, CHUNK), jnp.int32),
            pltpu.VMEM((K, CHUNK, EMB_DIM), jnp.float32),
            pltpu.SemaphoreType.DMA,
            pltpu.SemaphoreType.DMA,
        ],
    )
    return fn(idx, weight)


def kernel(token_ids, weight):
    idx = token_ids.reshape(NW, N_CHUNKS, CHUNK).astype(jnp.int32)
    out = _gather(idx, weight)
    return out.reshape(token_ids.shape[0], token_ids.shape[1], EMB_DIM)


# speedup vs baseline: 1.3071x; 1.3071x over previous
import functools

import jax
import jax.numpy as jnp
from jax import lax
from jax.experimental import pallas as pl
from jax.experimental.pallas import tpu as pltpu
from jax.experimental.pallas import tpu_sc as plsc

NUM_TOKENS = 16384 * 50
EMB_DIM = 32

NC = 2
NS = 16
NW = NC * NS
PER_W = NUM_TOKENS // NW
CHUNK = 128
N_CHUNKS = PER_W // CHUNK
K = 20
N_GROUPS = N_CHUNKS // K


def _body(idx_hbm, w_hbm, out_hbm, idx_v, bufs, sem_g, sem_o):
    wid = lax.axis_index("s") * NC + lax.axis_index("c")
    pltpu.sync_copy(idx_hbm.at[wid], idx_v)

    def group(m, carry):
        base = m * K
        gathers = [
            pltpu.async_copy(w_hbm.at[idx_v.at[base + b]], bufs.at[b], sem_g)
            for b in range(K)
        ]
        for cp in gathers:
            cp.wait()
        outs = [
            pltpu.async_copy(bufs.at[b], out_hbm.at[wid, base + b], sem_o)
            for b in range(K)
        ]
        for cp in outs:
            cp.wait()
        return carry

    lax.fori_loop(0, N_GROUPS, group, 0)


@jax.jit
def _gather(idx, weight):
    mesh = plsc.VectorSubcoreMesh(core_axis_name="c", subcore_axis_name="s")
    fn = pl.kernel(
        _body,
        out_type=jax.ShapeDtypeStruct((NW, N_CHUNKS, CHUNK, EMB_DIM), jnp.float32),
        mesh=mesh,
        scratch_types=[
            pltpu.VMEM((N_CHUNKS, CHUNK), jnp.int32),
            pltpu.VMEM((K, CHUNK, EMB_DIM), jnp.float32),
            pltpu.SemaphoreType.DMA,
            pltpu.SemaphoreType.DMA,
        ],
        compiler_params=pltpu.CompilerParams(use_tc_tiling_on_sc=False),
    )
    return fn(idx, weight)


def kernel(token_ids, weight):
    idx = token_ids.reshape(NW, N_CHUNKS, CHUNK).astype(jnp.int32)
    out = _gather(idx, weight)
    return out.reshape(token_ids.shape[0], token_ids.shape[1], EMB_DIM)


# speedup vs baseline: 1.7692x; 1.3535x over previous
import jax
import jax.numpy as jnp
from jax import lax
from jax.experimental import pallas as pl
from jax.experimental.pallas import tpu as pltpu
from jax.experimental.pallas import tpu_sc as plsc

N_ROWS = 16384
N_COLS = 50
EMB_DIM = 32

NC = 2
NS = 16
NW = NC * NS
ROWS_W = N_ROWS // NW
K = 16
N_GROUPS = ROWS_W // K


def _body(idx_hbm, w_hbm, out_hbm, idx_v, bufs, sem_g, sem_o):
    wid = lax.axis_index("s") * NC + lax.axis_index("c")
    row0 = wid * ROWS_W
    pltpu.sync_copy(idx_hbm.at[pl.ds(row0, ROWS_W)], idx_v)

    def group(m, carry):
        base = m * K
        gathers = [
            pltpu.async_copy(w_hbm.at[idx_v.at[base + r]], bufs.at[r], sem_g)
            for r in range(K)
        ]
        for cp in gathers:
            cp.wait()
        pltpu.async_copy(bufs, out_hbm.at[pl.ds(row0 + base, K)], sem_o).wait()
        return carry

    lax.fori_loop(0, N_GROUPS, group, 0)


@jax.jit
def _gather(idx, weight):
    mesh = plsc.VectorSubcoreMesh(core_axis_name="c", subcore_axis_name="s")
    fn = pl.kernel(
        _body,
        out_type=jax.ShapeDtypeStruct((N_ROWS, N_COLS, EMB_DIM), jnp.float32),
        mesh=mesh,
        scratch_types=[
            pltpu.VMEM((ROWS_W, N_COLS), jnp.int32),
            pltpu.VMEM((K, N_COLS, EMB_DIM), jnp.float32),
            pltpu.SemaphoreType.DMA,
            pltpu.SemaphoreType.DMA,
        ],
        compiler_params=pltpu.CompilerParams(use_tc_tiling_on_sc=False),
    )
    return fn(idx, weight)


def kernel(token_ids, weight):
    return _gather(token_ids.astype(jnp.int32), weight)
